# Initial kernel scaffold; baseline (speedup 1.0000x reference)
import jax
import jax.numpy as jnp
from jax.experimental import pallas as pl


def kernel(x, table, W, b):
    raise NotImplementedError("write your pallas kernel here")


# speedup vs baseline: 11.4270x; 11.4270x over previous
import functools

import jax
import jax.numpy as jnp
from jax import lax
from jax.experimental import pallas as pl
from jax.experimental.pallas import tpu as pltpu
from jax.experimental.pallas import tpu_sc as plsc

VOCAB = 100000
D_IN = 768
D_OUT = 128
BATCH = 4096
SEQ = 50
N_TOK = BATCH * SEQ

NC = 2
NS = 16
NW = NC * NS
PER_W = N_TOK // NW
CHUNK = 128
N_CH = PER_W // CHUNK

ROWS_BLK = 2000


def _proj_body(t_ref, w_ref, b_ref, o_ref):
    o_ref[...] = (
        jnp.dot(t_ref[...], w_ref[...], preferred_element_type=jnp.float32)
        + b_ref[...]
    )


def _project_table(table, W, b2d):
    return pl.pallas_call(
        _proj_body,
        grid=(VOCAB // ROWS_BLK,),
        in_specs=[
            pl.BlockSpec((ROWS_BLK, D_IN), lambda i: (i, 0)),
            pl.BlockSpec((D_IN, D_OUT), lambda i: (0, 0)),
            pl.BlockSpec((1, D_OUT), lambda i: (0, 0)),
        ],
        out_specs=pl.BlockSpec((ROWS_BLK, D_OUT), lambda i: (i, 0)),
        out_shape=jax.ShapeDtypeStruct((VOCAB, D_OUT), jnp.float32),
    )(table, W, b2d)


def _gather_body(pt_hbm, idx_hbm, out_hbm, idx_v, rows_v, gsem):
    wid = lax.axis_index("s") * NC + lax.axis_index("c")
    base = wid * PER_W
    pltpu.sync_copy(idx_hbm.at[wid], idx_v)

    def body(i, carry):
        pltpu.async_copy(pt_hbm.at[idx_v.at[i]], rows_v, gsem).wait()
        pltpu.sync_copy(rows_v, out_hbm.at[pl.ds(base + i * CHUNK, CHUNK)])
        return carry

    lax.fori_loop(0, N_CH, body, 0)


def _gather(pt, idx3):
    mesh = plsc.VectorSubcoreMesh(
        core_axis_name="c", subcore_axis_name="s", num_cores=NC, num_subcores=NS
    )
    k = functools.partial(
        pl.kernel,
        out_type=jax.ShapeDtypeStruct((N_TOK, D_OUT), jnp.float32),
        mesh=mesh,
        scratch_types=[
            pltpu.VMEM((N_CH, CHUNK), jnp.int32),
            pltpu.VMEM((CHUNK, D_OUT), jnp.float32),
            pltpu.SemaphoreType.DMA,
        ],
    )(_gather_body)
    return k(pt, idx3)


def kernel(x, table, W, b):
    pt = _project_table(table, W, b.reshape(1, D_OUT))
    idx3 = x.reshape(NW, N_CH, CHUNK).astype(jnp.int32)
    out = _gather(pt, idx3)
    return out.reshape(BATCH, SEQ, D_OUT)


# speedup vs baseline: 17.9182x; 1.5681x over previous
import functools

import jax
import jax.numpy as jnp
from jax import lax
from jax.experimental import pallas as pl
from jax.experimental.pallas import tpu as pltpu
from jax.experimental.pallas import tpu_sc as plsc

VOCAB = 100000
D_IN = 768
D_OUT = 128
BATCH = 4096
SEQ = 50

NC = 2
NS = 16
NW = NC * NS
B_PER_W = BATCH // NW
GRP = 4
N_GRP = B_PER_W // GRP

ROWS_BLK = 2000


def _proj_body(t_ref, w_ref, b_ref, o_ref):
    o_ref[...] = (
        jnp.dot(t_ref[...], w_ref[...], preferred_element_type=jnp.float32)
        + b_ref[...]
    )


def _project_table(table, W, b2d):
    return pl.pallas_call(
        _proj_body,
        grid=(VOCAB // ROWS_BLK,),
        in_specs=[
            pl.BlockSpec((ROWS_BLK, D_IN), lambda i: (i, 0)),
            pl.BlockSpec((D_IN, D_OUT), lambda i: (0, 0)),
            pl.BlockSpec((1, D_OUT), lambda i: (0, 0)),
        ],
        out_specs=pl.BlockSpec((ROWS_BLK, D_OUT), lambda i: (i, 0)),
        out_shape=jax.ShapeDtypeStruct((VOCAB, D_OUT), jnp.float32),
    )(table, W, b2d)


def _gather_body(pt_hbm, idx_hbm, out_hbm, idx_v, rows_v, gsem0, gsem1,
                 wsem0, wsem1):
    gsems = (gsem0, gsem1)
    wsems = (wsem0, wsem1)
    wid = lax.axis_index("s") * NC + lax.axis_index("c")
    b0 = wid * B_PER_W
    pltpu.sync_copy(idx_hbm.at[pl.ds(b0, B_PER_W)], idx_v)

    def _write_desc(g, p):
        return pltpu.make_async_copy(
            rows_v.at[pl.ds(p * GRP, GRP)],
            out_hbm.at[pl.ds(b0 + g * GRP, GRP)],
            wsems[p],
        )

    def body(i, carry):
        gathers = []
        for p in range(2):
            g = 2 * i + p

            @pl.when(i > 0)
            def _():
                _write_desc(g - 2, p).wait()

            for j in range(GRP):
                gathers.append(pltpu.async_copy(
                    pt_hbm.at[idx_v.at[g * GRP + j]],
                    rows_v.at[p * GRP + j],
                    gsems[p],
                ))
        for p in range(2):
            g = 2 * i + p
            for j in range(GRP):
                gathers[p * GRP + j].wait()
            _write_desc(g, p).start()
        return carry

    lax.fori_loop(0, N_GRP // 2, body, 0)
    _write_desc(N_GRP - 2, 0).wait()
    _write_desc(N_GRP - 1, 1).wait()


def _gather(pt, idx):
    mesh = plsc.VectorSubcoreMesh(
        core_axis_name="c", subcore_axis_name="s", num_cores=NC, num_subcores=NS
    )
    k = functools.partial(
        pl.kernel,
        out_type=jax.ShapeDtypeStruct((BATCH, SEQ, D_OUT), jnp.float32),
        mesh=mesh,
        scratch_types=[
            pltpu.VMEM((B_PER_W, SEQ), jnp.int32),
            pltpu.VMEM((2 * GRP, SEQ, D_OUT), jnp.float32),
            pltpu.SemaphoreType.DMA,
            pltpu.SemaphoreType.DMA,
            pltpu.SemaphoreType.DMA,
            pltpu.SemaphoreType.DMA,
        ],
    )(_gather_body)
    return k(pt, idx)


def kernel(x, table, W, b):
    pt = _project_table(table, W, b.reshape(1, D_OUT))
    return _gather(pt, x.astype(jnp.int32))
